# baseline (speedup 1.0000x reference)
import functools

import jax
import jax.numpy as jnp
from jax import lax
from jax.experimental import pallas as pl
from jax.experimental.pallas import tpu as pltpu
from jax.experimental.pallas import tpu_sc as plsc

N = 10000
E = 320000
D = 128

NC = 2
NS = 16
NW = NC * NS
EPW = E // NW
CH = 80
NCH = EPW // CH
SUP = 5
NSUP = NCH // SUP
SUPW = SUP * CH
RPT = N // NS


def _mesh():
    return plsc.VectorSubcoreMesh(
        core_axis_name="c", subcore_axis_name="s",
        num_cores=NC, num_subcores=NS)


def _sc_deg_body(dst_hbm, ew_hbm, out_hbm, deg_sh, dstv, ewv, zbuf,
                 dsem, esem):
    cid = lax.axis_index("c")
    sid = lax.axis_index("s")
    wid = sid * NC + cid
    z = jnp.zeros((16,), jnp.float32)

    @pl.loop(0, 64)
    def _zb(i):
        zbuf[pl.ds(i * 16, 16)] = z

    @pl.when(sid < 10)
    def _zero():
        pltpu.sync_copy(zbuf.at[pl.ds(0, 1000)],
                        deg_sh.at[pl.ds(sid * 1000, 1000)])

    plsc.subcore_barrier()

    pltpu.async_copy(dst_hbm.at[wid], dstv, dsem)
    pltpu.async_copy(ew_hbm.at[pl.ds(wid * EPW, EPW)], ewv, esem)
    pltpu.make_async_copy(dst_hbm.at[wid], dstv, dsem).wait()
    pltpu.make_async_copy(ew_hbm.at[pl.ds(wid * EPW, EPW)], ewv, esem).wait()

    def _sc_issue(c):
        pltpu.async_copy(ewv.at[pl.ds(c * CH, CH)],
                         deg_sh.at[dstv.at[c // SUP, c % SUP]], esem,
                         add=True)

    def _sc_drain(c):
        pltpu.make_async_copy(ewv.at[pl.ds(c * CH, CH)],
                              deg_sh.at[dstv.at[c // SUP, c % SUP]],
                              esem).wait()

    @pl.loop(0, NCH // SUP)
    def _chunk(g):
        for u in range(SUP):
            _sc_issue(g * SUP + u)
        for u in range(SUP):
            _sc_drain(g * SUP + u)

    plsc.subcore_barrier()

    @pl.when(sid < 10)
    def _drain():
        pltpu.sync_copy(deg_sh.at[pl.ds(sid * 1000, 1000)],
                        zbuf.at[pl.ds(0, 1000)])
        pltpu.sync_copy(zbuf.at[pl.ds(0, 1000)],
                        out_hbm.at[pl.ds(cid * N + sid * 1000, 1000)])


_sc_deg = functools.partial(
    pl.kernel,
    out_type=jax.ShapeDtypeStruct((NC * N,), jnp.float32),
    mesh=_mesh(),
    scratch_types=[
        pltpu.VMEM_SHARED((N,), jnp.float32),
        pltpu.VMEM((NSUP, SUP, CH), jnp.int32),
        pltpu.VMEM((EPW,), jnp.float32),
        pltpu.VMEM((1024,), jnp.float32),
        pltpu.SemaphoreType.DMA,
        pltpu.SemaphoreType.DMA,
    ],
)(_sc_deg_body)


def _tc_dense_body(pdegt_ref, ins_ref, w_ref, xs_ref, dis_ref):
    deg = jnp.sum(pdegt_ref[...], axis=1, keepdims=True) + 1.0
    dis = jnp.where(deg > 0, lax.rsqrt(deg), 0.0)
    dis_ref[...] = dis
    xs_ref[...] = dis * jnp.dot(ins_ref[...], w_ref[...],
                                preferred_element_type=jnp.float32)


def _tc_dense(pdegt, ins, W):
    return pl.pallas_call(
        _tc_dense_body,
        out_shape=(jax.ShapeDtypeStruct((N, D), jnp.float32),
                   jax.ShapeDtypeStruct((N, 1), jnp.float32)),
    )(pdegt, ins, W)


def _sc_edge_body(src_hbm, dst_hbm, ew_hbm, x_hbm, out_hbm,
                  acc, srcv, dstv, ewv, r0b, r1b, r2b, r3b,
                  g0, g1, g2, g3, s0, s1, s2, s3, st_src, st_ew, st_dst):
    cid = lax.axis_index("c")
    sid = lax.axis_index("s")
    wid = sid * NC + cid
    z = jnp.zeros((16,), jnp.float32)
    R = (r0b, r1b, r2b, r3b)
    G = (g0, g1, g2, g3)
    S = (s0, s1, s2, s3)

    @pl.loop(0, CH)
    def _zr(k):
        for j in range(8):
            r0b[k, pl.ds(j * 16, 16)] = z

    rbase = sid * 624

    @pl.loop(0, 7)
    def _za(i):
        pltpu.sync_copy(r0b, acc.at[pl.ds(rbase + i * CH, CH)])

    pltpu.sync_copy(r0b.at[pl.ds(0, 64)], acc.at[pl.ds(rbase + 560, 64)])

    @pl.when(sid == NS - 1)
    def _za_tail():
        pltpu.sync_copy(r0b.at[pl.ds(0, 16)], acc.at[pl.ds(9984, 16)])

    plsc.subcore_barrier()

    def stage_issue(p, slot):
        off = wid * EPW + p * SUPW
        pltpu.async_copy(src_hbm.at[pl.ds(off, SUPW)],
                         srcv.at[pl.ds(slot * SUPW, SUPW)], st_src)
        pltpu.async_copy(ew_hbm.at[pl.ds(off, SUPW)],
                         ewv.at[pl.ds(slot * SUPW, SUPW)], st_ew)
        pltpu.async_copy(dst_hbm.at[wid, p], dstv.at[slot], st_dst)

    def stage_wait(p, slot):
        off = wid * EPW + p * SUPW
        pltpu.make_async_copy(src_hbm.at[pl.ds(off, SUPW)],
                              srcv.at[pl.ds(slot * SUPW, SUPW)],
                              st_src).wait()
        pltpu.make_async_copy(ew_hbm.at[pl.ds(off, SUPW)],
                              ewv.at[pl.ds(slot * SUPW, SUPW)],
                              st_ew).wait()
        pltpu.make_async_copy(dst_hbm.at[wid, p], dstv.at[slot],
                              st_dst).wait()

    def sidx(c):
        return ((c // SUP) % 3) * SUPW + (c % SUP) * CH

    def didx(c):
        return dstv.at[(c // SUP) % 3, c % SUP]

    def g_issue(c, b):
        pltpu.async_copy(x_hbm.at[srcv.at[pl.ds(sidx(c), CH)]], R[b], G[b])

    def g_wait(c, b):
        pltpu.make_async_copy(x_hbm.at[srcv.at[pl.ds(sidx(c), CH)]],
                              R[b], G[b]).wait()

    def s_issue(c, b):
        pltpu.async_copy(R[b], acc.at[didx(c)], S[b], add=True)

    def s_wait(c, b):
        pltpu.make_async_copy(R[b], acc.at[didx(c)], S[b]).wait()

    def scale(c, b):
        rb = R[b]

        @plsc.parallel_loop(0, CH // 16, 1)
        def _norm(i):
            base = sidx(c) + i * 16
            for k2 in range(16):
                kk = i * 16 + k2
                w16 = plsc.load_gather(
                    ewv, [jnp.full((16,), base + k2, jnp.int32)])
                for j in range(8):
                    rb[kk, pl.ds(j * 16, 16)] = (
                        rb[kk, pl.ds(j * 16, 16)] * w16)

    def chunk_body(c, b, bn, peel=False):
        @pl.when(jnp.logical_and(c % SUP == 2, c // SUP + 1 <= NSUP - 1))
        def _stw():
            stage_wait(c // SUP + 1, (c // SUP + 1) % 3)

        g_wait(c, b)
        scale(c, b)
        s_issue(c, b)

        if peel:
            g_issue(c + 3, bn)
        else:
            @pl.when(c + 3 <= NCH - 1)
            def _pf():
                s_wait(c - 1, bn)
                g_issue(c + 3, bn)

        @pl.when(jnp.logical_and(c % SUP == 4, c // SUP + 2 <= NSUP - 1))
        def _st():
            stage_issue(c // SUP + 2, (c // SUP + 2) % 3)

    stage_issue(0, 0)
    stage_wait(0, 0)
    stage_issue(1, 1)
    g_issue(0, 0)
    g_issue(1, 1)
    g_issue(2, 2)
    chunk_body(0, 0, 3, peel=True)

    @pl.loop(0, 30)
    def _main(g):
        for u in range(4):
            c = 1 + g * 4 + u
            chunk_body(c, (1 + u) % 4, u)

    chunk_body(121, 1, 0)
    for c, b in ((122, 2), (123, 3)):
        g_wait(c, b)
        scale(c, b)
        s_issue(c, b)
    g_wait(124, 0)
    scale(124, 0)
    pltpu.sync_copy(R[0], acc.at[didx(124)], add=True)
    s_wait(121, 1)
    s_wait(122, 2)
    s_wait(123, 3)

    plsc.subcore_barrier()

    @pl.loop(0, 7)
    def _dr(i):
        pltpu.sync_copy(acc.at[pl.ds(rbase + i * CH, CH)], r0b)
        pltpu.sync_copy(r0b, out_hbm.at[cid, pl.ds(rbase + i * CH, CH)])

    pltpu.sync_copy(acc.at[pl.ds(rbase + 560, 64)], r0b.at[pl.ds(0, 64)])
    pltpu.sync_copy(r0b.at[pl.ds(0, 64)],
                    out_hbm.at[cid, pl.ds(rbase + 560, 64)])

    @pl.when(sid == NS - 1)
    def _dr_tail():
        pltpu.sync_copy(acc.at[pl.ds(9984, 16)], r0b.at[pl.ds(0, 16)])
        pltpu.sync_copy(r0b.at[pl.ds(0, 16)],
                        out_hbm.at[cid, pl.ds(9984, 16)])


_sc_edge = functools.partial(
    pl.kernel,
    out_type=jax.ShapeDtypeStruct((NC, N, D), jnp.float32),
    mesh=_mesh(),
    scratch_types=[
        pltpu.VMEM_SHARED((N, D), jnp.float32),
        pltpu.VMEM((3 * SUPW,), jnp.int32),
        pltpu.VMEM((3, SUP, CH), jnp.int32),
        pltpu.VMEM((3 * SUPW,), jnp.float32),
        pltpu.VMEM((CH, D), jnp.float32),
        pltpu.VMEM((CH, D), jnp.float32),
        pltpu.VMEM((CH, D), jnp.float32),
        pltpu.VMEM((CH, D), jnp.float32),
    ] + [pltpu.SemaphoreType.DMA] * 11,
    compiler_params=pltpu.CompilerParams(needs_layout_passes=False),
)(_sc_edge_body)


def _tc_final_body(p_ref, xs_ref, dis_ref, b_ref, o_ref):
    dis = dis_ref[...]
    agg = p_ref[0] + p_ref[1] + xs_ref[...]
    o_ref[...] = jnp.maximum(dis * agg + b_ref[...], 0.0)


def _tc_final(p, xs, dis_col, b2):
    return pl.pallas_call(
        _tc_final_body,
        out_shape=jax.ShapeDtypeStruct((N, D), jnp.float32),
    )(p, xs, dis_col, b2)


def kernel(ins, edge_index, edge_attr, W, b):
    src = edge_index[0]
    dst = edge_index[1].reshape(NW, NSUP, SUP, CH)
    ew = edge_attr
    pdegt = _sc_deg(dst, ew).reshape(NC, N).T
    xs, dis_col = _tc_dense(pdegt, ins, W)
    p = _sc_edge(src, dst, ew, xs)
    return _tc_final(p, xs, dis_col, b.reshape(1, D))


# speedup vs baseline: 1.0131x; 1.0131x over previous
import functools

import jax
import jax.numpy as jnp
from jax import lax
from jax.experimental import pallas as pl
from jax.experimental.pallas import tpu as pltpu
from jax.experimental.pallas import tpu_sc as plsc

N = 10000
E = 320000
D = 128

NC = 2
NS = 16
NW = NC * NS
EPW = E // NW
CH = 80
NCH = EPW // CH
SUP = 5
NSUP = NCH // SUP
SUPW = SUP * CH
RPT = N // NS


def _mesh():
    return plsc.VectorSubcoreMesh(
        core_axis_name="c", subcore_axis_name="s",
        num_cores=NC, num_subcores=NS)


def _sc_deg_body(dst_hbm, ew_hbm, out_hbm, deg_sh, dstv, ewv, zbuf,
                 dsem, esem):
    cid = lax.axis_index("c")
    sid = lax.axis_index("s")
    wid = sid * NC + cid
    z = jnp.zeros((16,), jnp.float32)

    @pl.loop(0, 64)
    def _zb(i):
        zbuf[pl.ds(i * 16, 16)] = z

    @pl.when(sid < 10)
    def _zero():
        pltpu.sync_copy(zbuf.at[pl.ds(0, 1000)],
                        deg_sh.at[pl.ds(sid * 1000, 1000)])

    plsc.subcore_barrier()

    pltpu.async_copy(dst_hbm.at[wid], dstv, dsem)
    pltpu.async_copy(ew_hbm.at[pl.ds(wid * EPW, EPW)], ewv, esem)
    pltpu.make_async_copy(dst_hbm.at[wid], dstv, dsem).wait()
    pltpu.make_async_copy(ew_hbm.at[pl.ds(wid * EPW, EPW)], ewv, esem).wait()

    def _sc_issue(c):
        pltpu.async_copy(ewv.at[pl.ds(c * CH, CH)],
                         deg_sh.at[dstv.at[c // SUP, c % SUP]], esem,
                         add=True)

    def _sc_drain(c):
        pltpu.make_async_copy(ewv.at[pl.ds(c * CH, CH)],
                              deg_sh.at[dstv.at[c // SUP, c % SUP]],
                              esem).wait()

    @pl.loop(0, NCH // SUP)
    def _chunk(g):
        for u in range(SUP):
            _sc_issue(g * SUP + u)
        for u in range(SUP):
            _sc_drain(g * SUP + u)

    plsc.subcore_barrier()

    @pl.when(sid < 10)
    def _drain():
        pltpu.sync_copy(deg_sh.at[pl.ds(sid * 1000, 1000)],
                        zbuf.at[pl.ds(0, 1000)])
        pltpu.sync_copy(zbuf.at[pl.ds(0, 1000)],
                        out_hbm.at[pl.ds(cid * N + sid * 1000, 1000)])


_sc_deg = functools.partial(
    pl.kernel,
    out_type=jax.ShapeDtypeStruct((NC * N,), jnp.float32),
    mesh=_mesh(),
    scratch_types=[
        pltpu.VMEM_SHARED((N,), jnp.float32),
        pltpu.VMEM((NSUP, SUP, CH), jnp.int32),
        pltpu.VMEM((EPW,), jnp.float32),
        pltpu.VMEM((1024,), jnp.float32),
        pltpu.SemaphoreType.DMA,
        pltpu.SemaphoreType.DMA,
    ],
)(_sc_deg_body)


def _tc_dense_body(pdegt_ref, ins_ref, w_ref, xs_ref, dis_ref):
    deg = jnp.sum(pdegt_ref[...], axis=1, keepdims=True) + 1.0
    dis = jnp.where(deg > 0, lax.rsqrt(deg), 0.0)
    dis_ref[...] = dis
    xs_ref[...] = dis * jnp.dot(ins_ref[...], w_ref[...],
                                preferred_element_type=jnp.float32)


def _tc_dense(pdegt, ins, W):
    return pl.pallas_call(
        _tc_dense_body,
        out_shape=(jax.ShapeDtypeStruct((N, D), jnp.float32),
                   jax.ShapeDtypeStruct((N, 1), jnp.float32)),
    )(pdegt, ins, W)


def _sc_edge_body(src_hbm, dst_hbm, ew_hbm, x_hbm, out_hbm,
                  acc, srcv, dstv, ewv, r0b, r1b, r2b, r3b,
                  g0, g1, g2, g3, s0, s1, s2, s3, st_src, st_ew, st_dst):
    cid = lax.axis_index("c")
    sid = lax.axis_index("s")
    wid = sid * NC + cid
    z = jnp.zeros((16,), jnp.float32)
    R = (r0b, r1b, r2b, r3b)
    G = (g0, g1, g2, g3)
    S = (s0, s1, s2, s3)

    def stage_issue(p, slot):
        off = wid * EPW + p * SUPW
        pltpu.async_copy(src_hbm.at[pl.ds(off, SUPW)],
                         srcv.at[pl.ds(slot * SUPW, SUPW)], st_src)
        pltpu.async_copy(ew_hbm.at[pl.ds(off, SUPW)],
                         ewv.at[pl.ds(slot * SUPW, SUPW)], st_ew)
        pltpu.async_copy(dst_hbm.at[wid, p], dstv.at[slot], st_dst)

    def stage_wait(p, slot):
        off = wid * EPW + p * SUPW
        pltpu.make_async_copy(src_hbm.at[pl.ds(off, SUPW)],
                              srcv.at[pl.ds(slot * SUPW, SUPW)],
                              st_src).wait()
        pltpu.make_async_copy(ew_hbm.at[pl.ds(off, SUPW)],
                              ewv.at[pl.ds(slot * SUPW, SUPW)],
                              st_ew).wait()
        pltpu.make_async_copy(dst_hbm.at[wid, p], dstv.at[slot],
                              st_dst).wait()


    stage_issue(0, 0)
    stage_issue(1, 1)

    @pl.loop(0, CH)
    def _zr(k):
        for j in range(8):
            r0b[k, pl.ds(j * 16, 16)] = z

    rbase = sid * 624

    @pl.loop(0, 7)
    def _za(i):
        pltpu.sync_copy(r0b, acc.at[pl.ds(rbase + i * CH, CH)])

    pltpu.sync_copy(r0b.at[pl.ds(0, 64)], acc.at[pl.ds(rbase + 560, 64)])

    @pl.when(sid == NS - 1)
    def _za_tail():
        pltpu.sync_copy(r0b.at[pl.ds(0, 16)], acc.at[pl.ds(9984, 16)])

    def sidx(c):
        return ((c // SUP) % 3) * SUPW + (c % SUP) * CH

    def didx(c):
        return dstv.at[(c // SUP) % 3, c % SUP]

    def g_issue(c, b):
        pltpu.async_copy(x_hbm.at[srcv.at[pl.ds(sidx(c), CH)]], R[b], G[b])

    def g_wait(c, b):
        pltpu.make_async_copy(x_hbm.at[srcv.at[pl.ds(sidx(c), CH)]],
                              R[b], G[b]).wait()

    def s_issue(c, b):
        pltpu.async_copy(R[b], acc.at[didx(c)], S[b], add=True)

    def s_wait(c, b):
        pltpu.make_async_copy(R[b], acc.at[didx(c)], S[b]).wait()

    def scale(c, b):
        rb = R[b]

        @plsc.parallel_loop(0, CH // 16, 1)
        def _norm(i):
            n16 = ewv[pl.ds(sidx(c) + i * 16, 16)]
            for k2 in range(16):
                w = n16[k2]
                kk = i * 16 + k2
                for j in range(8):
                    rb[kk, pl.ds(j * 16, 16)] = rb[kk, pl.ds(j * 16, 16)] * w

    def chunk_body(c, b, bn, peel=False):
        @pl.when(jnp.logical_and(
                jnp.logical_and(c % SUP == 2, c // SUP >= 1),
                c // SUP + 1 <= NSUP - 1))
        def _stw():
            stage_wait(c // SUP + 1, (c // SUP + 1) % 3)

        g_wait(c, b)
        scale(c, b)
        s_issue(c, b)

        if peel:
            g_issue(c + 3, bn)
        else:
            @pl.when(c + 3 <= NCH - 1)
            def _pf():
                s_wait(c - 1, bn)
                g_issue(c + 3, bn)

        @pl.when(jnp.logical_and(c % SUP == 4, c // SUP + 2 <= NSUP - 1))
        def _st():
            stage_issue(c // SUP + 2, (c // SUP + 2) % 3)

    stage_wait(0, 0)
    stage_wait(1, 1)
    g_issue(0, 0)
    g_issue(1, 1)
    g_issue(2, 2)
    plsc.subcore_barrier()
    chunk_body(0, 0, 3, peel=True)

    @pl.loop(0, 30)
    def _main(g):
        for u in range(4):
            c = 1 + g * 4 + u
            chunk_body(c, (1 + u) % 4, u)

    chunk_body(121, 1, 0)
    for c, b in ((122, 2), (123, 3)):
        g_wait(c, b)
        scale(c, b)
        s_issue(c, b)
    g_wait(124, 0)
    scale(124, 0)
    pltpu.sync_copy(R[0], acc.at[didx(124)], add=True)
    s_wait(121, 1)
    s_wait(122, 2)
    s_wait(123, 3)

    plsc.subcore_barrier()

    @pl.loop(0, 7)
    def _dr(i):
        pltpu.sync_copy(acc.at[pl.ds(rbase + i * CH, CH)], r0b)
        pltpu.sync_copy(r0b, out_hbm.at[cid, pl.ds(rbase + i * CH, CH)])

    pltpu.sync_copy(acc.at[pl.ds(rbase + 560, 64)], r0b.at[pl.ds(0, 64)])
    pltpu.sync_copy(r0b.at[pl.ds(0, 64)],
                    out_hbm.at[cid, pl.ds(rbase + 560, 64)])

    @pl.when(sid == NS - 1)
    def _dr_tail():
        pltpu.sync_copy(acc.at[pl.ds(9984, 16)], r0b.at[pl.ds(0, 16)])
        pltpu.sync_copy(r0b.at[pl.ds(0, 16)],
                        out_hbm.at[cid, pl.ds(9984, 16)])


_sc_edge = functools.partial(
    pl.kernel,
    out_type=jax.ShapeDtypeStruct((NC, N, D), jnp.float32),
    mesh=_mesh(),
    scratch_types=[
        pltpu.VMEM_SHARED((N, D), jnp.float32),
        pltpu.VMEM((3 * SUPW,), jnp.int32),
        pltpu.VMEM((3, SUP, CH), jnp.int32),
        pltpu.VMEM((3 * SUPW,), jnp.float32),
        pltpu.VMEM((CH, D), jnp.float32),
        pltpu.VMEM((CH, D), jnp.float32),
        pltpu.VMEM((CH, D), jnp.float32),
        pltpu.VMEM((CH, D), jnp.float32),
    ] + [pltpu.SemaphoreType.DMA] * 11,
)(_sc_edge_body)


def _tc_final_body(p_ref, xs_ref, dis_ref, b_ref, o_ref):
    dis = dis_ref[...]
    agg = p_ref[0] + p_ref[1] + xs_ref[...]
    o_ref[...] = jnp.maximum(dis * agg + b_ref[...], 0.0)


def _tc_final(p, xs, dis_col, b2):
    return pl.pallas_call(
        _tc_final_body,
        out_shape=jax.ShapeDtypeStruct((N, D), jnp.float32),
    )(p, xs, dis_col, b2)


def kernel(ins, edge_index, edge_attr, W, b):
    src = edge_index[0]
    dst = edge_index[1].reshape(NW, NSUP, SUP, CH)
    ew = edge_attr
    pdegt = _sc_deg(dst, ew).reshape(NC, N).T
    xs, dis_col = _tc_dense(pdegt, ins, W)
    p = _sc_edge(src, dst, ew, xs)
    return _tc_final(p, xs, dis_col, b.reshape(1, D))
